# Initial kernel scaffold; baseline (speedup 1.0000x reference)
import jax
import jax.numpy as jnp
from jax.experimental import pallas as pl


def kernel(x, edge_index, edge_attr, batch, params):
    raise NotImplementedError("write your pallas kernel here")


# speedup vs baseline: 1.9559x; 1.9559x over previous
import functools

import jax
import jax.numpy as jnp
from jax import lax
from jax.experimental import pallas as pl
from jax.experimental.pallas import tpu as pltpu
from jax.experimental.pallas import tpu_sc as plsc

N = 10000
E = 320000
DE = 16
H = 128
OUT = 10
G = 128

_EB = 2000


def _edge_mlp_body(ea_ref, w1_ref, b1_ref, w2_ref, b2_ref, out_ref):
    h = jnp.maximum(
        jnp.dot(ea_ref[...], w1_ref[...], preferred_element_type=jnp.float32)
        + b1_ref[...], 0.0)
    out_ref[...] = (
        jnp.dot(h, w2_ref[...], preferred_element_type=jnp.float32)
        + b2_ref[...])


def _edge_mlp(edge_attr, w1, b1, w2, b2):
    grid = (E // _EB,)
    return pl.pallas_call(
        _edge_mlp_body,
        grid=grid,
        in_specs=[
            pl.BlockSpec((_EB, DE), lambda i: (i, 0)),
            pl.BlockSpec((DE, H), lambda i: (0, 0)),
            pl.BlockSpec((1, H), lambda i: (0, 0)),
            pl.BlockSpec((H, H), lambda i: (0, 0)),
            pl.BlockSpec((1, H), lambda i: (0, 0)),
        ],
        out_specs=pl.BlockSpec((_EB, H), lambda i: (i, 0)),
        out_shape=jax.ShapeDtypeStruct((E, H), jnp.float32),
    )(edge_attr, w1, b1.reshape(1, H), w2, b2.reshape(1, H))


_NC = 2
_NS = 16
_NW = _NC * _NS
_EPW = E // _NW
_CH = 80
_NCHUNK = _EPW // _CH
_ZR = 200
_NZCH = N // _ZR
_ZK = (_NZCH + _NS - 1) // _NS

@functools.lru_cache(maxsize=None)
def _build_sc_agg():
  mesh = plsc.VectorSubcoreMesh(core_axis_name="c", subcore_axis_name="s")

  @functools.partial(
    pl.kernel,
    out_type=jax.ShapeDtypeStruct((_NC, N, H), jnp.float32),
    mesh=mesh,
    scratch_types=[
        pltpu.VMEM((_CH,), jnp.int32),
        pltpu.VMEM((_CH,), jnp.int32),
        pltpu.VMEM((_CH,), jnp.int32),
        pltpu.VMEM((_CH, H), jnp.float32),
        pltpu.VMEM((_CH, H), jnp.float32),
        pltpu.VMEM((_ZR, H), jnp.float32),
        pltpu.VMEM_SHARED((N, H), jnp.float32),
        pltpu.SemaphoreType.DMA,
    ],
  )
  def sc_agg(x_hbm, e_hbm, src_hbm, dst_hbm, ord_hbm, out_hbm,
             ordv, srcv, dstv, xrows, erows, zbuf, aggs, sem):
    cid = lax.axis_index("c")
    sid = lax.axis_index("s")
    wid = sid * _NC + cid

    zero16 = jnp.zeros((16,), jnp.float32)

    def _zero_row(r, _):
        for c in range(H // 16):
            zbuf[r, pl.ds(c * 16, 16)] = zero16
        return 0

    lax.fori_loop(0, _ZR, _zero_row, 0)
    for k in range(_ZK):
        idx = sid + _NS * k

        @pl.when(idx < _NZCH)
        def _():
            pltpu.sync_copy(zbuf, aggs.at[pl.ds(idx * _ZR, _ZR)])
    plsc.subcore_barrier()

    def _chunk(i, _):
        base = wid * _EPW + i * _CH
        pltpu.sync_copy(ord_hbm.at[pl.ds(base, _CH)], ordv)
        pltpu.async_copy(src_hbm.at[ordv], srcv, sem).wait()
        pltpu.async_copy(dst_hbm.at[ordv], dstv, sem).wait()
        pltpu.async_copy(e_hbm.at[ordv], erows, sem).wait()
        pltpu.async_copy(x_hbm.at[srcv], xrows, sem).wait()

        def _row(r, _):
            for c in range(H // 16):
                sl = pl.ds(c * 16, 16)
                xrows[r, sl] = jnp.maximum(xrows[r, sl] + erows[r, sl], 0.0)
            return 0

        lax.fori_loop(0, _CH, _row, 0)
        pltpu.sync_copy(xrows, aggs.at[dstv], add=True)
        return 0

    lax.fori_loop(0, _NCHUNK, _chunk, 0)
    plsc.subcore_barrier()
    for k in range(_ZK):
        idx = sid + _NS * k

        @pl.when(idx < _NZCH)
        def _():
            off = idx * _ZR
            pltpu.sync_copy(aggs.at[pl.ds(off, _ZR)],
                            out_hbm.at[cid, pl.ds(off, _ZR)])

  return sc_agg


def _sc_agg_call(x, e, src, dst, order):
    return _build_sc_agg()(x, e, src, dst, order)


_NB = 2000


def _node_mlp_body(x_ref, a0_ref, a1_ref, eps_ref, m1_ref, c1_ref,
                   m2_ref, c2_ref, t_ref, st_ref, acc_ref):
    i = pl.program_id(0)
    h = (1.0 + eps_ref[0, 0]) * x_ref[...] + (a0_ref[...] + a1_ref[...])
    u = jnp.maximum(
        jnp.dot(h, m1_ref[...], preferred_element_type=jnp.float32)
        + c1_ref[...], 0.0)
    t = jnp.maximum(
        jnp.dot(u, m2_ref[...], preferred_element_type=jnp.float32)
        + c2_ref[...], 0.0)
    t_ref[...] = t

    @pl.when(i == 0)
    def _():
        acc_ref[...] = jnp.zeros_like(acc_ref)

    acc_ref[...] += jnp.sum(t, axis=0, keepdims=True)

    @pl.when(i == pl.num_programs(0) - 1)
    def _():
        st_ref[...] = acc_ref[...]


def _node_mlp(x, a0, a1, eps, m1, c1, m2, c2):
    grid = (N // _NB,)
    return pl.pallas_call(
        _node_mlp_body,
        grid=grid,
        in_specs=[
            pl.BlockSpec((_NB, H), lambda i: (i, 0)),
            pl.BlockSpec((_NB, H), lambda i: (i, 0)),
            pl.BlockSpec((_NB, H), lambda i: (i, 0)),
            pl.BlockSpec((1, 1), lambda i: (0, 0)),
            pl.BlockSpec((H, H), lambda i: (0, 0)),
            pl.BlockSpec((1, H), lambda i: (0, 0)),
            pl.BlockSpec((H, H), lambda i: (0, 0)),
            pl.BlockSpec((1, H), lambda i: (0, 0)),
        ],
        out_specs=[
            pl.BlockSpec((_NB, H), lambda i: (i, 0)),
            pl.BlockSpec((1, H), lambda i: (0, 0)),
        ],
        out_shape=[
            jax.ShapeDtypeStruct((N, H), jnp.float32),
            jax.ShapeDtypeStruct((1, H), jnp.float32),
        ],
        scratch_shapes=[pltpu.VMEM((1, H), jnp.float32)],
    )(x, a0, a1, eps.reshape(1, 1), m1, c1.reshape(1, H), m2,
      c2.reshape(1, H))


def _bn_body(t_ref, st_ref, g_ref, b_ref, out_ref, vacc_ref):
    p = pl.program_id(0)
    i = pl.program_id(1)
    mean = st_ref[...] / N

    @pl.when(p == 0)
    def _():
        @pl.when(i == 0)
        def _():
            vacc_ref[...] = jnp.zeros_like(vacc_ref)
        d = t_ref[...] - mean
        vacc_ref[...] += jnp.sum(d * d, axis=0, keepdims=True)

    @pl.when(p == 1)
    def _():
        var = vacc_ref[...] / N
        out_ref[...] = ((t_ref[...] - mean) / jnp.sqrt(var + 1e-5)
                        * g_ref[...] + b_ref[...])


def _bn_apply(t, st, gamma, beta):
    grid = (2, N // _NB)
    return pl.pallas_call(
        _bn_body,
        grid=grid,
        in_specs=[
            pl.BlockSpec((_NB, H), lambda p, i: (i, 0)),
            pl.BlockSpec((1, H), lambda p, i: (0, 0)),
            pl.BlockSpec((1, H), lambda p, i: (0, 0)),
            pl.BlockSpec((1, H), lambda p, i: (0, 0)),
        ],
        out_specs=pl.BlockSpec((_NB, H), lambda p, i: (i, 0)),
        out_shape=jax.ShapeDtypeStruct((N, H), jnp.float32),
        scratch_shapes=[pltpu.VMEM((1, H), jnp.float32)],
    )(t, st, gamma.reshape(1, H), beta.reshape(1, H))


def _pool_body(h0_ref, h1_ref, h2_ref, h3_ref, b_ref, w1_ref, b1_ref,
               w4_ref, b4_ref, out_ref, sum_ref, cnt_ref):
    i = pl.program_id(0)

    @pl.when(i == 0)
    def _():
        sum_ref[...] = jnp.zeros_like(sum_ref)
        cnt_ref[...] = jnp.zeros_like(cnt_ref)

    bids = b_ref[0]
    gids = lax.broadcasted_iota(jnp.int32, (G, _NB), 0)
    oh = (gids == bids).astype(jnp.float32)
    z = jnp.concatenate(
        [h0_ref[...], h1_ref[...], h2_ref[...], h3_ref[...]], axis=1)
    sum_ref[...] += jnp.dot(oh, z, preferred_element_type=jnp.float32,
                            precision=lax.Precision.HIGHEST)
    cnt_ref[...] += jnp.sum(oh, axis=1, keepdims=True)

    @pl.when(i == pl.num_programs(0) - 1)
    def _():
        pooled = sum_ref[...] / jnp.maximum(cnt_ref[...], 1.0)
        o = jnp.maximum(
            jnp.dot(pooled, w1_ref[...], preferred_element_type=jnp.float32)
            + b1_ref[...], 0.0)
        out_ref[...] = (
            jnp.dot(o, w4_ref[...], preferred_element_type=jnp.float32)
            + b4_ref[...])


def _pool_head(hs, batch, w1, b1, w4, b4):
    grid = (N // _NB,)
    batch_r = batch.reshape(N // _NB, 1, _NB)
    return pl.pallas_call(
        _pool_body,
        grid=grid,
        in_specs=[
            pl.BlockSpec((_NB, H), lambda i: (i, 0)),
            pl.BlockSpec((_NB, H), lambda i: (i, 0)),
            pl.BlockSpec((_NB, H), lambda i: (i, 0)),
            pl.BlockSpec((_NB, H), lambda i: (i, 0)),
            pl.BlockSpec((1, 1, _NB), lambda i: (i, 0, 0)),
            pl.BlockSpec((4 * H, H), lambda i: (0, 0)),
            pl.BlockSpec((1, H), lambda i: (0, 0)),
            pl.BlockSpec((H, OUT), lambda i: (0, 0)),
            pl.BlockSpec((1, OUT), lambda i: (0, 0)),
        ],
        out_specs=pl.BlockSpec((G, OUT), lambda i: (0, 0)),
        out_shape=jax.ShapeDtypeStruct((G, OUT), jnp.float32),
        scratch_shapes=[
            pltpu.VMEM((G, 4 * H), jnp.float32),
            pltpu.VMEM((G, 1), jnp.float32),
        ],
    )(hs[0], hs[1], hs[2], hs[3], batch_r, w1, b1.reshape(1, H), w4,
      b4.reshape(1, OUT))


def kernel(x, edge_index, edge_attr, batch, params):
    src = edge_index[0]
    dst = edge_index[1]
    order = jnp.argsort(dst).astype(jnp.int32)
    hs = []
    h = x
    for c in range(4):
        p = params[f"conv{c}"]
        e = _edge_mlp(edge_attr, p["be1"][0], p["be1"][1],
                      p["be2"][0], p["be2"][1])
        aggp = _sc_agg_call(h, e, src, dst, order)
        t, st = _node_mlp(h, aggp[0], aggp[1], p["eps"],
                          p["mlp1"][0], p["mlp1"][1],
                          p["mlp2"][0], p["mlp2"][1])
        bn = params[f"bn{c}"]
        h = _bn_apply(t, st, bn["gamma"], bn["beta"])
        hs.append(h)
    return _pool_head(hs, batch, params["fc1"][0], params["fc1"][1],
                      params["fc4"][0], params["fc4"][1])
